# baseline (speedup 1.0000x reference)
import jax
import jax.numpy as jnp
from jax.experimental import pallas as pl


def _add_pe_kernel(x_ref, pe_ref, o_ref):
    o_ref[...] = x_ref[...] + pe_ref[...][None, :, :]


def kernel(x, pos_table):
    B, S, D = x.shape
    Sb = 512
    grid = (S // Sb,)
    return pl.pallas_call(
        _add_pe_kernel,
        grid=grid,
        in_specs=[
            pl.BlockSpec((B, Sb, D), lambda s: (0, s, 0)),
            pl.BlockSpec((Sb, D), lambda s: (s, 0)),
        ],
        out_specs=pl.BlockSpec((B, Sb, D), lambda s: (0, s, 0)),
        out_shape=jax.ShapeDtypeStruct((B, S, D), x.dtype),
    )(x, pos_table)
